# baseline (speedup 1.0000x reference)
import functools

import jax
import jax.numpy as jnp
from jax import lax
from jax.experimental import pallas as pl
from jax.experimental.pallas import tpu as pltpu
from jax.experimental.pallas import tpu_sc as plsc

D_MODEL = 128
N_BINS = 8
N_COMB = N_BINS ** 3
BATCH = 16384
HIST = 50
PAD_H = 56
IDX_PAD = 64
IDX_COLS = 128
CIDX_ROWS = BATCH * IDX_PAD // IDX_COLS

N_WORKERS = 32
B_PER_W = BATCH // N_WORKERS
NB = 8
N_CHUNKS = B_PER_W // NB


def _table_body(embS_ref, embU_ref, embF_ref, out_ref):
    c = lax.broadcasted_iota(jnp.int32, (N_COMB, N_BINS), 0)
    j = lax.broadcasted_iota(jnp.int32, (N_COMB, N_BINS), 1)
    ohS = jnp.where((c >> 6) == j, 1.0, 0.0)
    ohU = jnp.where(((c >> 3) & 7) == j, 1.0, 0.0)
    ohF = jnp.where((c & 7) == j, 1.0, 0.0)
    out_ref[...] = (
        jnp.dot(ohS, embS_ref[...], preferred_element_type=jnp.float32)
        + jnp.dot(ohU, embU_ref[...], preferred_element_type=jnp.float32)
        + jnp.dot(ohF, embF_ref[...], preferred_element_type=jnp.float32)
    )


def _build_table(embS, embU, embF):
    return pl.pallas_call(
        _table_body,
        out_shape=jax.ShapeDtypeStruct((N_COMB, D_MODEL), jnp.float32),
    )(embS, embU, embF)


def _cidx_body(s_ref, u_ref, f_ref, o_ref):
    o_ref[...] = s_ref[...] * 64 + u_ref[...] * 8 + f_ref[...]


def _combine_idx(binS, binU, binF):
    def pad(x):
        return jnp.pad(x, ((0, 0), (0, IDX_PAD - HIST))).reshape(
            CIDX_ROWS, IDX_COLS)
    grid = 8
    blk = CIDX_ROWS // grid
    spec = pl.BlockSpec((blk, IDX_COLS), lambda i: (i, 0))
    return pl.pallas_call(
        _cidx_body,
        grid=(grid,),
        in_specs=[spec, spec, spec],
        out_specs=spec,
        out_shape=jax.ShapeDtypeStruct((CIDX_ROWS, IDX_COLS), jnp.int32),
    )(pad(binS), pad(binU), pad(binF))


_mesh = plsc.VectorSubcoreMesh(core_axis_name="c", subcore_axis_name="s")


@functools.partial(
    pl.kernel,
    mesh=_mesh,
    out_type=jax.ShapeDtypeStruct((BATCH * PAD_H, D_MODEL), jnp.float32),
    scratch_types=[
        pltpu.VMEM((NB, IDX_PAD), jnp.int32),
        pltpu.VMEM((NB * IDX_PAD, D_MODEL), jnp.float32),
        pltpu.SemaphoreType.DMA,
        pltpu.SemaphoreType.DMA,
    ],
)
def _sc_lookup(table_hbm, cidx_hbm, out_hbm, idx_v, rows_v, sem_g, sem_s):
    wid = lax.axis_index("s") * 2 + lax.axis_index("c")
    b_base = wid * B_PER_W

    def body(i, carry):
        b0 = b_base + i * NB
        pltpu.sync_copy(cidx_hbm.at[pl.ds(b0, NB)], idx_v)
        gathers = []
        for j in range(NB):
            gathers.append(pltpu.async_copy(
                table_hbm.at[idx_v.at[j]],
                rows_v.at[pl.ds(j * IDX_PAD, IDX_PAD)],
                sem_g,
            ))
        for cp in gathers:
            cp.wait()
        scatters = []
        for j in range(NB):
            scatters.append(pltpu.async_copy(
                rows_v.at[pl.ds(j * IDX_PAD, PAD_H)],
                out_hbm.at[pl.ds((b0 + j) * PAD_H, PAD_H)],
                sem_s,
            ))
        for cp in scatters:
            cp.wait()
        return carry

    lax.fori_loop(0, N_CHUNKS, body, 0)


def kernel(binS, binU, binF, embS, embU, embF):
    table = _build_table(embS, embU, embF)
    cidx = _combine_idx(binS, binU, binF).reshape(BATCH, IDX_PAD)
    out = _sc_lookup(table, cidx)
    return out.reshape(BATCH, PAD_H, D_MODEL)[:, :HIST, :]


# speedup vs baseline: 1.9675x; 1.9675x over previous
import functools

import jax
import jax.numpy as jnp
from jax import lax
from jax.experimental import pallas as pl
from jax.experimental.pallas import tpu as pltpu
from jax.experimental.pallas import tpu_sc as plsc

D_MODEL = 128
N_BINS = 8
N_COMB = N_BINS ** 3
BATCH = 16384
HIST = 50
PAD_H = 56
IDX_COLS = 128
CIDX_ROWS = BATCH * PAD_H // IDX_COLS
N_PAD_ROWS = BATCH * PAD_H

N_WORKERS = 32
B_PER_W = BATCH // N_WORKERS
NB = 16
CHUNK = NB * PAD_H
G_SUB = CHUNK // IDX_COLS
N_CHUNKS = B_PER_W // NB


def _table_body(embS_ref, embU_ref, embF_ref, out_ref):
    c = lax.broadcasted_iota(jnp.int32, (N_COMB, N_BINS), 0)
    j = lax.broadcasted_iota(jnp.int32, (N_COMB, N_BINS), 1)
    ohS = jnp.where((c >> 6) == j, 1.0, 0.0)
    ohU = jnp.where(((c >> 3) & 7) == j, 1.0, 0.0)
    ohF = jnp.where((c & 7) == j, 1.0, 0.0)
    out_ref[...] = (
        jnp.dot(ohS, embS_ref[...], preferred_element_type=jnp.float32)
        + jnp.dot(ohU, embU_ref[...], preferred_element_type=jnp.float32)
        + jnp.dot(ohF, embF_ref[...], preferred_element_type=jnp.float32)
    )


def _build_table(embS, embU, embF):
    return pl.pallas_call(
        _table_body,
        out_shape=jax.ShapeDtypeStruct((N_COMB, D_MODEL), jnp.float32),
    )(embS, embU, embF)


def _cidx_body(s_ref, u_ref, f_ref, o_ref):
    o_ref[...] = s_ref[...] * 64 + u_ref[...] * 8 + f_ref[...]


def _combine_idx(binS, binU, binF):
    def pad(x):
        return jnp.pad(x, ((0, 0), (0, PAD_H - HIST))).reshape(
            CIDX_ROWS, IDX_COLS)
    grid = 8
    blk = CIDX_ROWS // grid
    spec = pl.BlockSpec((blk, IDX_COLS), lambda i: (i, 0))
    return pl.pallas_call(
        _cidx_body,
        grid=(grid,),
        in_specs=[spec, spec, spec],
        out_specs=spec,
        out_shape=jax.ShapeDtypeStruct((CIDX_ROWS, IDX_COLS), jnp.int32),
    )(pad(binS), pad(binU), pad(binF))


_mesh = plsc.VectorSubcoreMesh(core_axis_name="c", subcore_axis_name="s")


@functools.partial(
    pl.kernel,
    mesh=_mesh,
    out_type=jax.ShapeDtypeStruct((N_PAD_ROWS, D_MODEL), jnp.float32),
    scratch_types=[
        pltpu.VMEM((CHUNK,), jnp.int32),
        pltpu.VMEM((CHUNK, D_MODEL), jnp.float32),
        pltpu.SemaphoreType.DMA,
    ],
)
def _sc_lookup(table_hbm, cidx_hbm, out_hbm, idx_v, rows_v, sem_g):
    wid = lax.axis_index("s") * 2 + lax.axis_index("c")
    row_base = wid * B_PER_W * PAD_H

    def body(i, carry):
        r0 = row_base + i * CHUNK
        pltpu.sync_copy(cidx_hbm.at[pl.ds(r0, CHUNK)], idx_v)
        gathers = []
        for j in range(G_SUB):
            gathers.append(pltpu.async_copy(
                table_hbm.at[idx_v.at[pl.ds(j * IDX_COLS, IDX_COLS)]],
                rows_v.at[pl.ds(j * IDX_COLS, IDX_COLS)],
                sem_g,
            ))
        for cp in gathers:
            cp.wait()
        pltpu.sync_copy(rows_v, out_hbm.at[pl.ds(r0, CHUNK)])
        return carry

    lax.fori_loop(0, N_CHUNKS, body, 0)


def kernel(binS, binU, binF, embS, embU, embF):
    table = _build_table(embS, embU, embF)
    cidx = _combine_idx(binS, binU, binF).reshape(N_PAD_ROWS)
    out = _sc_lookup(table, cidx)
    return out.reshape(BATCH, PAD_H, D_MODEL)[:, :HIST, :]
